# baseline (speedup 1.0000x reference)
import functools

import jax
import jax.numpy as jnp
from jax.experimental import pallas as pl
from jax.experimental.pallas import tpu as pltpu

_NEG_SLOPE = 0.01
_NEG = -1e30


def _leaky(z):
    return jnp.where(z > 0, z, _NEG_SLOPE * z)


def _pool_sel(p_rows, q_rows, wo, wi, valid_h, valid_w):
    p = jnp.arange(p_rows)[:, None]
    q = jnp.arange(q_rows)[None, :]
    i, j = p // wo, p % wo
    valid = (i < valid_h) & (j < valid_w)
    target = jnp.where(valid, 2 * wi * i + 2 * j, -1)
    return (q == target).astype(jnp.bfloat16)


def _pool_flat(z, *, ws, valid_w, masked_edges):
    m, c = z.shape
    r = jax.lax.broadcasted_iota(jnp.int32, (m, c), 0)
    w = r % ws
    if valid_w < ws:
        z = jnp.where(w < valid_w, z, _NEG)
    neg_row = jnp.full((1, c), _NEG, jnp.float32)
    sd = jnp.concatenate([neg_row, z[:-1]], axis=0)
    su = jnp.concatenate([z[1:], neg_row], axis=0)
    if masked_edges:
        sd = jnp.where(w > 0, sd, _NEG)
        su = jnp.where(w < ws - 1, su, _NEG)
    m1 = jnp.maximum(jnp.maximum(sd, su), z)
    neg_band = jnp.full((ws, c), _NEG, jnp.float32)
    md = jnp.concatenate([neg_band, m1[:-ws]], axis=0)
    mu = jnp.concatenate([m1[ws:], neg_band], axis=0)
    return jnp.maximum(jnp.maximum(md, mu), m1)


def _c1_body(p_ref, w_ref, b_ref, s_ref, o_ref):
    z = jax.lax.dot_general(p_ref[0], w_ref[...], (((0,), (0,)), ((), ())),
                            preferred_element_type=jnp.float32)
    z = _leaky(z + b_ref[...])
    m2 = _pool_flat(z, ws=56, valid_w=56, masked_edges=True)
    o_ref[0] = jnp.dot(s_ref[...], m2.astype(jnp.bfloat16),
                       preferred_element_type=jnp.float32).astype(jnp.bfloat16)


def _conv1_pool(patches, w, b, sel):
    n, k, m = patches.shape
    cout = w.shape[1]
    pr = sel.shape[0]
    return pl.pallas_call(
        _c1_body,
        out_shape=jax.ShapeDtypeStruct((n, pr, cout), jnp.bfloat16),
        grid=(n,),
        in_specs=[
            pl.BlockSpec((1, k, m), lambda i: (i, 0, 0)),
            pl.BlockSpec((k, cout), lambda i: (0, 0)),
            pl.BlockSpec((1, cout), lambda i: (0, 0)),
            pl.BlockSpec((pr, m), lambda i: (0, 0)),
        ],
        out_specs=pl.BlockSpec((1, pr, cout), lambda i: (i, 0, 0)),
        compiler_params=pltpu.CompilerParams(
            dimension_semantics=("parallel",)),
    )(patches, w, b, sel)


def _conv_body(x_ref, w_ref, b_ref, s_ref, o_ref, *, offsets, m, ws, valid_w):
    x = x_ref[0]
    acc = jnp.dot(x[offsets[0]:offsets[0] + m, :], w_ref[0],
                  preferred_element_type=jnp.float32)
    for t in range(1, len(offsets)):
        off = offsets[t]
        acc = acc + jnp.dot(x[off:off + m, :], w_ref[t],
                            preferred_element_type=jnp.float32)
    z = _leaky(acc + b_ref[...])
    m2 = _pool_flat(z, ws=ws, valid_w=valid_w, masked_edges=False)
    o_ref[0] = jnp.dot(s_ref[...], m2.astype(jnp.bfloat16),
                       preferred_element_type=jnp.float32).astype(jnp.bfloat16)


def _conv_pool(x, w, b, sel, *, kh, kw, ws, m, valid_w):
    n, rows, cin = x.shape
    taps, _, cout = w.shape
    assert taps == kh * kw
    pr = sel.shape[0]
    offsets = tuple(dh * ws + dw for dh in range(kh) for dw in range(kw))
    return pl.pallas_call(
        functools.partial(_conv_body, offsets=offsets, m=m, ws=ws,
                          valid_w=valid_w),
        out_shape=jax.ShapeDtypeStruct((n, pr, cout), jnp.bfloat16),
        grid=(n,),
        in_specs=[
            pl.BlockSpec((1, rows, cin), lambda i: (i, 0, 0)),
            pl.BlockSpec((taps, cin, cout), lambda i: (0, 0, 0)),
            pl.BlockSpec((1, cout), lambda i: (0, 0)),
            pl.BlockSpec((pr, m), lambda i: (0, 0)),
        ],
        out_specs=pl.BlockSpec((1, pr, cout), lambda i: (i, 0, 0)),
        compiler_params=pltpu.CompilerParams(
            dimension_semantics=("parallel",)),
    )(x, w, b, sel)


def _fc_body(a_ref, w1_ref, b1_ref, w2_ref, b2_ref, w3_ref, b3_ref, o_ref,
             *, valid_cols):
    h1 = _leaky(jnp.dot(a_ref[...], w1_ref[...],
                        preferred_element_type=jnp.float32) + b1_ref[...])
    h2 = _leaky(jnp.dot(h1.astype(jnp.bfloat16), w2_ref[...],
                        preferred_element_type=jnp.float32) + b2_ref[...])
    z = jnp.dot(h2.astype(jnp.bfloat16), w3_ref[...],
                preferred_element_type=jnp.float32) + b3_ref[...]
    col = jax.lax.broadcasted_iota(jnp.int32, z.shape, 1)
    valid = col < valid_cols
    zm = jnp.where(valid, z, -jnp.inf)
    mx = jnp.max(zm, axis=-1, keepdims=True)
    e = jnp.where(valid, jnp.exp(z - mx), 0.0)
    lse = jnp.log(jnp.sum(e, axis=-1, keepdims=True)) + mx
    o_ref[...] = z - lse


def _fc_fused(a, w1, b1, w2, b2, w3, b3, *, valid_cols):
    m, k1 = a.shape
    n1, n2, n3 = w1.shape[1], w2.shape[1], w3.shape[1]
    full = lambda shape: pl.BlockSpec(shape, lambda: tuple(0 for _ in shape))
    return pl.pallas_call(
        functools.partial(_fc_body, valid_cols=valid_cols),
        out_shape=jax.ShapeDtypeStruct((m, n3), jnp.float32),
        in_specs=[
            full((m, k1)),
            full((k1, n1)), full((1, n1)),
            full((n1, n2)), full((1, n2)),
            full((n2, n3)), full((1, n3)),
        ],
        out_specs=full((m, n3)),
        compiler_params=pltpu.CompilerParams(
            vmem_limit_bytes=100 * 1024 * 1024),
    )(a, w1, b1, w2, b2, w3, b3)


def kernel(conv1_w, conv1_b, conv2_w, conv2_b, conv3_w, conv3_b,
           fc1_w, fc1_b, fc2_w, fc2_b, fc3_w, fc3_b, x_nchw):
    bf = jnp.bfloat16
    n = x_nchw.shape[0]

    patches = jax.lax.conv_general_dilated_patches(
        x_nchw.astype(bf), filter_shape=(7, 7), window_strides=(4, 4),
        padding="VALID",
        dimension_numbers=("NCHW", "OIHW", "NCHW"))
    patches = patches.reshape(n, 147, 56 * 56)
    w1 = conv1_w.reshape(7, 7, 3, 128).transpose(2, 0, 1, 3).reshape(147, 128)

    s1 = _pool_sel(928, 3136, 32, 56, 28, 28)
    s2 = _pool_sel(208, 768, 16, 32, 12, 12)
    s3 = _pool_sel(25, 160, 5, 16, 5, 5)

    p1 = _conv1_pool(patches, w1.astype(bf), conv1_b, s1)
    p2 = _conv_pool(p1, conv2_w.astype(bf), conv2_b, s2,
                    kh=5, kw=5, ws=32, m=768, valid_w=24)
    p3 = _conv_pool(p2, conv3_w.astype(bf), conv3_b, s3,
                    kh=3, kw=3, ws=16, m=160, valid_w=10)

    a = p3.reshape(n, 9600)
    out = _fc_fused(a, fc1_w.astype(bf), fc1_b, fc2_w.astype(bf), fc2_b,
                    fc3_w.astype(bf), fc3_b, valid_cols=6)
    return out[:, :6]


# speedup vs baseline: 1.5953x; 1.5953x over previous
import functools

import jax
import jax.numpy as jnp
from jax.experimental import pallas as pl
from jax.experimental.pallas import tpu as pltpu

_NEG_SLOPE = 0.01
_NEG = -1e30


def _leaky(z):
    return jnp.where(z > 0, z, _NEG_SLOPE * z)


def _neg_like(shape, dtype):
    return jnp.full(shape, _NEG, dtype)


def _wmax3(z):
    r, q, c = z.shape
    pad = _neg_like((r, 64, c), z.dtype)
    sd = jnp.concatenate([pad, z[:, :-64, :]], axis=1)
    su = jnp.concatenate([z[:, 64:, :], pad], axis=1)
    return jnp.maximum(jnp.maximum(sd, su), z)


def _even_w_pick(m, wo):
    r, q, c = m.shape
    m4 = m.reshape(r, wo, 64, c)
    m4 = jnp.concatenate([m4[:, k:k + 1] for k in range(0, wo, 2)], axis=1)
    return m4.reshape(r, (wo // 2) * 64, c)


def _c1_body(a_ref, h_ref, w_ref, b_ref, o_ref):
    b0 = pl.program_id(0)

    def conv_rows(v):
        r = v.shape[0]
        z = jnp.dot(v.reshape(r * 3584, 147), w_ref[...],
                    preferred_element_type=jnp.float32)
        return _leaky(z + b_ref[...]).astype(jnp.bfloat16).reshape(r, 3584, 128)

    za = conv_rows(a_ref[...])
    zh = conv_rows(h_ref[...])
    zh = jnp.where(b0 == 0, jnp.bfloat16(_NEG), zh)
    m1 = _wmax3(jnp.concatenate([zh, za], axis=0))
    m2 = jnp.maximum(jnp.maximum(m1[0:1], m1[1:2]), m1[2:3])
    o_ref[...] = _even_w_pick(m2, 56)


def _conv1_pool(pt, w, b):
    return pl.pallas_call(
        _c1_body,
        out_shape=jax.ShapeDtypeStruct((28, 1792, 128), jnp.bfloat16),
        grid=(28,),
        in_specs=[
            pl.BlockSpec((2, 3584, 147), lambda i: (i, 0, 0)),
            pl.BlockSpec((1, 3584, 147),
                         lambda i: (jnp.maximum(2 * i - 1, 0), 0, 0)),
            pl.BlockSpec((147, 128), lambda i: (0, 0)),
            pl.BlockSpec((1, 128), lambda i: (0, 0)),
        ],
        out_specs=pl.BlockSpec((1, 1792, 128), lambda i: (i, 0, 0)),
        compiler_params=pltpu.CompilerParams(
            dimension_semantics=("parallel",),
            vmem_limit_bytes=64 * 1024 * 1024),
    )(pt, pt, w, b)


def _c2_body(*refs):
    xs_refs, w_ref, b_ref, o_ref = refs[:7], refs[7], refs[8], refs[9]
    t0 = pl.program_id(0)
    xs = jnp.concatenate([r[...] for r in xs_refs], axis=0)
    acc = None
    for t in range(25):
        dh, dw = t // 5, t % 5
        blk = xs[dh:dh + 3, 64 * dw:64 * dw + 1536, :].reshape(4608, 128)
        d = jnp.dot(blk, w_ref[t], preferred_element_type=jnp.float32)
        acc = d if acc is None else acc + d
    z = _leaky(acc + b_ref[...]).astype(jnp.bfloat16).reshape(3, 1536, 256)
    hix = jax.lax.broadcasted_iota(jnp.int32, (3, 1, 1), 0)
    z = jnp.where((t0 == 0) & (hix < 1), jnp.bfloat16(_NEG), z)
    m1 = _wmax3(z)
    m2 = jnp.maximum(jnp.maximum(m1[0:1], m1[1:2]), m1[2:3])
    o_ref[...] = _even_w_pick(m2, 24)


def _conv2_pool(p1, w, b):
    row_specs = [
        pl.BlockSpec((1, 1792, 128),
                     functools.partial(
                         lambda i, k=0: (jnp.clip(2 * i - 1 + k, 0, 27), 0, 0),
                         k=k))
        for k in range(7)
    ]
    return pl.pallas_call(
        _c2_body,
        out_shape=jax.ShapeDtypeStruct((12, 768, 256), jnp.bfloat16),
        grid=(12,),
        in_specs=row_specs + [
            pl.BlockSpec((25, 128, 256), lambda i: (0, 0, 0)),
            pl.BlockSpec((1, 256), lambda i: (0, 0)),
        ],
        out_specs=pl.BlockSpec((1, 768, 256), lambda i: (i, 0, 0)),
        compiler_params=pltpu.CompilerParams(
            dimension_semantics=("parallel",),
            vmem_limit_bytes=64 * 1024 * 1024),
    )(*([p1] * 7), w, b)


def _c3_body(*refs):
    xs_refs, w_ref, b_ref, o_ref = refs[:5], refs[5], refs[6], refs[7]
    t0 = pl.program_id(0)
    xs = jnp.concatenate([r[...] for r in xs_refs], axis=0)
    acc = None
    for t in range(9):
        dh, dw = t // 3, t % 3
        blk = xs[dh:dh + 3, 64 * dw:64 * dw + 640, :].reshape(1920, 256)
        d = jnp.dot(blk, w_ref[t], preferred_element_type=jnp.float32)
        acc = d if acc is None else acc + d
    z = _leaky(acc + b_ref[...]).astype(jnp.bfloat16).reshape(3, 640, 384)
    hix = jax.lax.broadcasted_iota(jnp.int32, (3, 1, 1), 0)
    z = jnp.where((t0 == 0) & (hix < 1), jnp.bfloat16(_NEG), z)
    m1 = _wmax3(z)
    m2 = jnp.maximum(jnp.maximum(m1[0:1], m1[1:2]), m1[2:3])
    o_ref[...] = _even_w_pick(m2, 10)


def _conv3_pool(p2, w, b):
    row_specs = [
        pl.BlockSpec((1, 768, 256),
                     functools.partial(
                         lambda i, k=0: (jnp.clip(2 * i - 1 + k, 0, 11), 0, 0),
                         k=k))
        for k in range(5)
    ]
    return pl.pallas_call(
        _c3_body,
        out_shape=jax.ShapeDtypeStruct((5, 320, 384), jnp.bfloat16),
        grid=(5,),
        in_specs=row_specs + [
            pl.BlockSpec((9, 256, 384), lambda i: (0, 0, 0)),
            pl.BlockSpec((1, 384), lambda i: (0, 0)),
        ],
        out_specs=pl.BlockSpec((1, 320, 384), lambda i: (i, 0, 0)),
        compiler_params=pltpu.CompilerParams(
            dimension_semantics=("parallel",),
            vmem_limit_bytes=64 * 1024 * 1024),
    )(*([p2] * 5), w, b)


def _fc_body(p3_ref, w1_ref, b1_ref, w2_ref, b2_ref, w3_ref, b3_ref, o_ref,
             *, valid_cols):
    acc = None
    for h in range(5):
        for w in range(5):
            slab = p3_ref[h, 64 * w:64 * w + 64, :]
            d = jnp.dot(slab, w1_ref[5 * h + w],
                        preferred_element_type=jnp.float32)
            acc = d if acc is None else acc + d
    h1 = _leaky(acc + b1_ref[...])
    h2 = _leaky(jnp.dot(h1.astype(jnp.bfloat16), w2_ref[...],
                        preferred_element_type=jnp.float32) + b2_ref[...])
    z = jnp.dot(h2.astype(jnp.bfloat16), w3_ref[...],
                preferred_element_type=jnp.float32) + b3_ref[...]
    col = jax.lax.broadcasted_iota(jnp.int32, z.shape, 1)
    valid = col < valid_cols
    zm = jnp.where(valid, z, -jnp.inf)
    mx = jnp.max(zm, axis=-1, keepdims=True)
    e = jnp.where(valid, jnp.exp(z - mx), 0.0)
    lse = jnp.log(jnp.sum(e, axis=-1, keepdims=True)) + mx
    o_ref[...] = z - lse


def _fc_fused(p3, w1, b1, w2, b2, w3, b3, *, valid_cols):
    full = lambda shape: pl.BlockSpec(shape, lambda: tuple(0 for _ in shape))
    return pl.pallas_call(
        functools.partial(_fc_body, valid_cols=valid_cols),
        out_shape=jax.ShapeDtypeStruct((64, 128), jnp.float32),
        in_specs=[
            full((5, 320, 384)),
            full((25, 384, 512)), full((1, 512)),
            full((512, 512)), full((1, 512)),
            full((512, 128)), full((1, 128)),
        ],
        out_specs=full((64, 128)),
        compiler_params=pltpu.CompilerParams(
            vmem_limit_bytes=64 * 1024 * 1024),
    )(p3, w1, b1, w2, b2, w3, b3)


def kernel(conv1_w, conv1_b, conv2_w, conv2_b, conv3_w, conv3_b,
           fc1_w, fc1_b, fc2_w, fc2_b, fc3_w, fc3_b, x_nchw):
    bf = jnp.bfloat16
    n = x_nchw.shape[0]

    patches = jax.lax.conv_general_dilated_patches(
        x_nchw.astype(bf), filter_shape=(7, 7), window_strides=(4, 4),
        padding="VALID",
        dimension_numbers=("NCHW", "OIHW", "NHWC"))
    pt = jnp.transpose(patches, (1, 2, 0, 3)).reshape(56, 56 * n, 147)
    w1 = conv1_w.reshape(7, 7, 3, 128).transpose(2, 0, 1, 3).reshape(147, 128)

    p1 = _conv1_pool(pt, w1.astype(bf), conv1_b)
    p2 = _conv2_pool(p1, conv2_w.astype(bf), conv2_b)
    p3 = _conv3_pool(p2, conv3_w.astype(bf), conv3_b)
    out = _fc_fused(p3, fc1_w.reshape(25, 384, 512).astype(bf), fc1_b,
                    fc2_w.astype(bf), fc2_b, fc3_w.astype(bf), fc3_b,
                    valid_cols=6)
    return out[:, :6]
